# Initial kernel scaffold; baseline (speedup 1.0000x reference)
import jax
import jax.numpy as jnp
from jax.experimental import pallas as pl


def kernel(x, edge_index, batch, W1, b1, W2, b2, Wh, bh):
    raise NotImplementedError("write your pallas kernel here")


# speedup vs baseline: 21.6939x; 21.6939x over previous
import functools

import jax
import jax.numpy as jnp
from jax import lax
from jax.experimental import pallas as pl
from jax.experimental.pallas import tpu as pltpu
from jax.experimental.pallas import tpu_sc as plsc

_NP = 10240
_G = 64
_NW = 32
_CHUNK = 128
_RPS = _NP // 16
_BLK = 2048
_NBLK = _NP // _BLK


def _sc_degree(dst3):
    K = dst3.shape[1]
    mesh = plsc.VectorSubcoreMesh(core_axis_name="c", subcore_axis_name="s")

    @functools.partial(
        pl.kernel,
        out_type=jax.ShapeDtypeStruct((2, _NP), jnp.float32),
        mesh=mesh,
        scratch_types=[
            pltpu.VMEM((K, _CHUNK), jnp.int32),
            pltpu.VMEM((_CHUNK,), jnp.float32),
            pltpu.VMEM((_RPS,), jnp.float32),
            pltpu.VMEM_SHARED((_NP,), jnp.float32),
        ],
        compiler_params=pltpu.CompilerParams(use_tc_tiling_on_sc=False),
    )
    def k(dst_hbm, out_hbm, idx_v, ones_v, row_v, deg_sh):
        cid = lax.axis_index("c")
        sid = lax.axis_index("s")
        wid = sid * 2 + cid
        pltpu.sync_copy(dst_hbm.at[wid], idx_v)
        for i in range(_CHUNK // 16):
            ones_v[pl.ds(i * 16, 16)] = jnp.ones((16,), jnp.float32)

        def zfill(i, c):
            row_v[pl.ds(i * 16, 16)] = jnp.zeros((16,), jnp.float32)
            return c

        lax.fori_loop(0, _RPS // 16, zfill, 0)
        pltpu.sync_copy(row_v, deg_sh.at[pl.ds(sid * _RPS, _RPS)])
        plsc.subcore_barrier()

        def body(j, c):
            pltpu.sync_copy(ones_v, deg_sh.at[idx_v.at[j]], add=True)
            return c

        lax.fori_loop(0, K, body, 0)
        plsc.subcore_barrier()
        pltpu.sync_copy(deg_sh.at[pl.ds(sid * _RPS, _RPS)], row_v)
        pltpu.sync_copy(row_v, out_hbm.at[cid, pl.ds(sid * _RPS, _RPS)])

    return k(dst3)


def _sc_aggregate(p, src3, dst3):
    K = src3.shape[1]
    mesh = plsc.VectorSubcoreMesh(core_axis_name="c", subcore_axis_name="s")

    @functools.partial(
        pl.kernel,
        out_type=jax.ShapeDtypeStruct((2, _NP, 64), jnp.float32),
        mesh=mesh,
        scratch_types=[
            pltpu.VMEM((K, _CHUNK), jnp.int32),
            pltpu.VMEM((K, _CHUNK), jnp.int32),
            pltpu.VMEM((_CHUNK, 64), jnp.float32),
            pltpu.VMEM((64, 64), jnp.float32),
            pltpu.VMEM_SHARED((_NP, 64), jnp.float32),
        ],
        compiler_params=pltpu.CompilerParams(use_tc_tiling_on_sc=False),
    )
    def k(p_hbm, src_hbm, dst_hbm, out_hbm, si_v, di_v, gbuf, zbuf, q_sh):
        cid = lax.axis_index("c")
        sid = lax.axis_index("s")
        wid = sid * 2 + cid
        pltpu.sync_copy(src_hbm.at[wid], si_v)
        pltpu.sync_copy(dst_hbm.at[wid], di_v)

        def zfill(i, c):
            for t in range(4):
                zbuf[i, pl.ds(t * 16, 16)] = jnp.zeros((16,), jnp.float32)
            return c

        lax.fori_loop(0, 64, zfill, 0)

        def zcopy(t, c):
            pltpu.sync_copy(zbuf, q_sh.at[pl.ds(sid * _RPS + t * 64, 64)])
            return c

        lax.fori_loop(0, _RPS // 64, zcopy, 0)
        plsc.subcore_barrier()

        def body(j, c):
            pltpu.sync_copy(p_hbm.at[si_v.at[j]], gbuf)
            pltpu.sync_copy(gbuf, q_sh.at[di_v.at[j]], add=True)
            return c

        lax.fori_loop(0, K, body, 0)
        plsc.subcore_barrier()

        def ocopy(t, c):
            r = sid * _RPS + t * _CHUNK
            pltpu.sync_copy(q_sh.at[pl.ds(r, _CHUNK)], gbuf)
            pltpu.sync_copy(gbuf, out_hbm.at[cid, pl.ds(r, _CHUNK)])
            return c

        lax.fori_loop(0, _RPS // _CHUNK, ocopy, 0)

    return k(p, src3, dst3)


def _tc_layer1(x_pad, degp, W1):
    def body(x_ref, d_ref, w_ref, o_ref):
        dinv = lax.rsqrt(d_ref[0] + d_ref[1] + 1.0)
        h = jnp.dot(x_ref[...], w_ref[...], preferred_element_type=jnp.float32)
        o_ref[...] = h * dinv[:, None]

    return pl.pallas_call(
        body,
        grid=(_NBLK,),
        in_specs=[
            pl.BlockSpec((_BLK, 128), lambda i: (i, 0)),
            pl.BlockSpec((2, _BLK), lambda i: (0, i)),
            pl.BlockSpec((128, 64), lambda i: (0, 0)),
        ],
        out_specs=pl.BlockSpec((_BLK, 64), lambda i: (i, 0)),
        out_shape=jax.ShapeDtypeStruct((_NP, 64), jnp.float32),
    )(x_pad, degp, W1)


def _tc_layer2(degp, qp, p1, W2, b1):
    def body(d_ref, q_ref, p_ref, w_ref, b_ref, o_ref):
        dinv = lax.rsqrt(d_ref[0] + d_ref[1] + 1.0)
        agg = q_ref[0] + q_ref[1] + p_ref[...]
        r = jnp.maximum(agg * dinv[:, None] + b_ref[...], 0.0)
        h2 = jnp.dot(r, w_ref[...], preferred_element_type=jnp.float32)
        o_ref[...] = h2 * dinv[:, None]

    return pl.pallas_call(
        body,
        grid=(_NBLK,),
        in_specs=[
            pl.BlockSpec((2, _BLK), lambda i: (0, i)),
            pl.BlockSpec((2, _BLK, 64), lambda i: (0, i, 0)),
            pl.BlockSpec((_BLK, 64), lambda i: (i, 0)),
            pl.BlockSpec((64, 64), lambda i: (0, 0)),
            pl.BlockSpec((1, 64), lambda i: (0, 0)),
        ],
        out_specs=pl.BlockSpec((_BLK, 64), lambda i: (i, 0)),
        out_shape=jax.ShapeDtypeStruct((_NP, 64), jnp.float32),
    )(degp, qp, p1, W2, b1)


def _tc_final(degp, qp, p2, b2, batch3, Wh, bh):
    def body(d_ref, q_ref, p_ref, b_ref, bat_ref, wh_ref, bh_ref, o_ref,
             acc_s, cnt_s):
        i = pl.program_id(0)

        @pl.when(i == 0)
        def _():
            acc_s[...] = jnp.zeros_like(acc_s)
            cnt_s[...] = jnp.zeros_like(cnt_s)

        dinv = lax.rsqrt(d_ref[0] + d_ref[1] + 1.0)
        agg = q_ref[0] + q_ref[1] + p_ref[...]
        r = jnp.maximum(agg * dinv[:, None] + b_ref[...], 0.0)
        gids = lax.broadcasted_iota(jnp.int32, (_G, _BLK), 0)
        m = (bat_ref[0] == gids).astype(jnp.float32)
        acc_s[...] += jnp.dot(m, r, preferred_element_type=jnp.float32)
        cnt_s[...] += jnp.sum(m, axis=1, keepdims=True)

        @pl.when(i == _NBLK - 1)
        def _():
            pooled = acc_s[...] / jnp.maximum(cnt_s[...], 1.0)
            o_ref[...] = (
                jnp.dot(pooled, wh_ref[...], preferred_element_type=jnp.float32)
                + bh_ref[...]
            )

    return pl.pallas_call(
        body,
        grid=(_NBLK,),
        in_specs=[
            pl.BlockSpec((2, _BLK), lambda i: (0, i)),
            pl.BlockSpec((2, _BLK, 64), lambda i: (0, i, 0)),
            pl.BlockSpec((_BLK, 64), lambda i: (i, 0)),
            pl.BlockSpec((1, 64), lambda i: (0, 0)),
            pl.BlockSpec((1, 1, _BLK), lambda i: (i, 0, 0)),
            pl.BlockSpec((64, 20), lambda i: (0, 0)),
            pl.BlockSpec((1, 20), lambda i: (0, 0)),
        ],
        out_specs=pl.BlockSpec((_G, 20), lambda i: (0, 0)),
        out_shape=jax.ShapeDtypeStruct((_G, 20), jnp.float32),
        scratch_shapes=[
            pltpu.VMEM((_G, 64), jnp.float32),
            pltpu.VMEM((_G, 1), jnp.float32),
        ],
    )(degp, qp, p2, b2, batch3, Wh, bh)


def kernel(x, edge_index, batch, W1, b1, W2, b2, Wh, bh):
    N, D = x.shape
    E = edge_index.shape[1]
    K = -(-E // (_NW * _CHUNK))
    EP = _NW * K * _CHUNK

    x_pad = jnp.zeros((_NP, D), x.dtype).at[:N].set(x)
    pad_idx = jnp.full((EP - E,), _NP - 1, jnp.int32)
    src3 = jnp.concatenate([edge_index[0], pad_idx]).reshape(_NW, K, _CHUNK)
    dst3 = jnp.concatenate([edge_index[1], pad_idx]).reshape(_NW, K, _CHUNK)
    bat_pad = jnp.concatenate(
        [batch, jnp.full((_NP - N,), _G, jnp.int32)]
    ).reshape(_NBLK, 1, _BLK)
    b1r = b1.reshape(1, -1)
    b2r = b2.reshape(1, -1)
    bhr = bh.reshape(1, -1)

    degp = _sc_degree(dst3)
    p1 = _tc_layer1(x_pad, degp, W1)
    q1 = _sc_aggregate(p1, src3, dst3)
    p2 = _tc_layer2(degp, q1, p1, W2, b1r)
    q2 = _sc_aggregate(p2, src3, dst3)
    return _tc_final(degp, q2, p2, b2r, bat_pad, Wh, bhr)
